# baseline (speedup 1.0000x reference)
import jax
import jax.numpy as jnp
from jax import lax
from jax.experimental import pallas as pl
from jax.experimental.pallas import tpu as pltpu
from jax.experimental.pallas import tpu_sc as plsc

VSZ = 1000000
DSZ = 64
B = 4096
L = 50

NC = 2
NS = 16
NW = NC * NS

TOTAL = B * L
PER_W = TOTAL // NW
CH = 128
NCHUNK = PER_W // CH
NBUF = 5

TBLK = 32768
NFULL = VSZ // TBLK
VPAD = (NFULL + 1) * TBLK
SH = (TBLK // 2).bit_length() - 1

BBLK = 256


def _tc_body(tT_ref, out_ref):
    a = tT_ref[...].T
    out_ref[:, :DSZ] = a[: TBLK // 2]
    out_ref[:, DSZ:] = a[TBLK // 2 :]


def _tc_body_tail(tail_ref, full_ref, out_ref):
    del full_ref
    a = tail_ref[...].T
    out_ref[:, :DSZ] = a[: TBLK // 2]
    out_ref[:, DSZ:] = a[TBLK // 2 :]


def _relayout(tT):
    out_shape = jax.ShapeDtypeStruct((VPAD // 2, 2 * DSZ), jnp.float32)
    full = pl.pallas_call(
        _tc_body,
        grid=(NFULL,),
        in_specs=[pl.BlockSpec((DSZ, TBLK), lambda i: (0, i))],
        out_specs=pl.BlockSpec((TBLK // 2, 2 * DSZ), lambda i: (i, 0)),
        out_shape=out_shape,
    )(tT)
    tail = jnp.pad(
        lax.slice(tT, (0, NFULL * TBLK), (DSZ, VSZ)),
        ((0, 0), (0, VPAD - VSZ)),
    )
    return pl.pallas_call(
        _tc_body_tail,
        grid=(1,),
        in_specs=[
            pl.BlockSpec((DSZ, TBLK), lambda i: (0, 0)),
            pl.BlockSpec(memory_space=pl.ANY),
        ],
        out_specs=pl.BlockSpec((TBLK // 2, 2 * DSZ), lambda i: (NFULL, 0)),
        out_shape=out_shape,
        input_output_aliases={1: 0},
    )(tail, full)


def _sc_body(idx_hbm, table_hbm, out_hbm, idx1_v, idx_v, rows_v, *sems):
    gsem = sems[:NBUF]
    ssem = sems[NBUF:]
    wid = lax.axis_index("s") * NC + lax.axis_index("c")
    base = wid * PER_W
    pltpu.sync_copy(idx_hbm.at[pl.ds(base, PER_W)], idx1_v)

    @pl.loop(0, NCHUNK)
    def _(c):
        for v in range(CH // 16):
            t = idx1_v[pl.ds(c * CH + v * 16, 16)]
            m = t & (TBLK - 1)
            j = (t - m) + ((m + m) & (TBLK - 1)) + (m >> SH)
            idx_v[c, pl.ds(v * 16, 16)] = j

    for b in range(NBUF):
        pltpu.async_copy(table_hbm.at[idx_v.at[b]], rows_v.at[b], gsem[b])

    @pl.loop(0, NCHUNK - NBUF, step=NBUF)
    def _(i):
        for b in range(NBUF):
            c = i + b
            pltpu.make_async_copy(
                table_hbm.at[idx_v.at[c]], rows_v.at[b], gsem[b]
            ).wait()
            pltpu.async_copy(
                rows_v.at[b], out_hbm.at[pl.ds(base + c * CH, CH)], ssem[b]
            )
        for b in range(NBUF):
            c = i + b
            pltpu.make_async_copy(
                rows_v.at[b], out_hbm.at[pl.ds(base + c * CH, CH)], ssem[b]
            ).wait()
            pltpu.async_copy(
                table_hbm.at[idx_v.at[c + NBUF]], rows_v.at[b], gsem[b]
            )

    for b in range(NBUF):
        c = NCHUNK - NBUF + b
        pltpu.make_async_copy(
            table_hbm.at[idx_v.at[c]], rows_v.at[b], gsem[b]
        ).wait()
        pltpu.async_copy(
            rows_v.at[b], out_hbm.at[pl.ds(base + c * CH, CH)], ssem[b]
        )
    for b in range(NBUF):
        c = NCHUNK - NBUF + b
        pltpu.make_async_copy(
            rows_v.at[b], out_hbm.at[pl.ds(base + c * CH, CH)], ssem[b]
        ).wait()


def _gather(idx, table_rm):
    mesh = plsc.VectorSubcoreMesh(core_axis_name="c", subcore_axis_name="s")
    return pl.kernel(
        _sc_body,
        out_type=jax.ShapeDtypeStruct((TOTAL, DSZ), jnp.float32),
        mesh=mesh,
        scratch_types=[
            pltpu.VMEM((PER_W,), jnp.int32),
            pltpu.VMEM((NCHUNK, CH), jnp.int32),
            pltpu.VMEM((NBUF, CH, DSZ), jnp.float32),
        ]
        + [pltpu.SemaphoreType.DMA] * (2 * NBUF),
        compiler_params=pltpu.CompilerParams(use_tc_tiling_on_sc=False),
    )(idx, table_rm)


def _tc_post_body(in_hbm, out_ref, buf, sem):
    i = pl.program_id(0)
    slot = lax.rem(i, 2)

    @pl.when(i == 0)
    def _():
        pltpu.make_async_copy(
            in_hbm.at[pl.ds(0, BBLK)], buf.at[0], sem.at[0]
        ).start()

    @pl.when(i + 1 < B // BBLK)
    def _():
        pltpu.make_async_copy(
            in_hbm.at[pl.ds((i + 1) * BBLK, BBLK)],
            buf.at[1 - slot],
            sem.at[1 - slot],
        ).start()

    pltpu.make_async_copy(
        in_hbm.at[pl.ds(i * BBLK, BBLK)], buf.at[slot], sem.at[slot]
    ).wait()

    out_ref[...] = buf[slot].T


def _final_transpose(flat):
    return pl.pallas_call(
        _tc_post_body,
        grid=(B // BBLK,),
        in_specs=[pl.BlockSpec(memory_space=pl.ANY)],
        out_specs=pl.BlockSpec((L * DSZ, BBLK), lambda i: (0, i)),
        out_shape=jax.ShapeDtypeStruct((L * DSZ, B), jnp.float32),
        scratch_shapes=[
            pltpu.VMEM((2, BBLK, L * DSZ), jnp.float32),
            pltpu.SemaphoreType.DMA((2,)),
        ],
        compiler_params=pltpu.CompilerParams(needs_layout_passes=False),
    )(flat)


@jax.jit
def _lookup(x, table):
    t2 = _relayout(table.T)
    table_rm = t2.reshape(VPAD, DSZ)
    idx = x.reshape(TOTAL)
    flat = _gather(idx, table_rm)
    post = _final_transpose(flat.reshape(B, L * DSZ))
    return jnp.transpose(post.reshape(L, DSZ, B), (2, 0, 1))


def kernel(x, table):
    return _lookup(x, table)


# speedup vs baseline: 1.0119x; 1.0119x over previous
import jax
import jax.numpy as jnp
from jax import lax
from jax.experimental import pallas as pl
from jax.experimental.pallas import tpu as pltpu
from jax.experimental.pallas import tpu_sc as plsc

VSZ = 1000000
DSZ = 64
B = 4096
L = 50

NC = 2
NS = 16
NW = NC * NS

TOTAL = B * L
HALF = TOTAL // 2
PER_W = HALF // NW
CH = 128
NCHUNK = PER_W // CH
NBUF = 5

TBLK = 32768
NFULL = VSZ // TBLK
VPAD = (NFULL + 1) * TBLK
SH = (TBLK // 2).bit_length() - 1

BBLK = 256


def _tc_body(tT_ref, out_ref):
    a = tT_ref[...].T
    out_ref[:, :DSZ] = a[: TBLK // 2]
    out_ref[:, DSZ:] = a[TBLK // 2 :]


def _tc_body_tail(tail_ref, full_ref, out_ref):
    del full_ref
    a = tail_ref[...].T
    out_ref[:, :DSZ] = a[: TBLK // 2]
    out_ref[:, DSZ:] = a[TBLK // 2 :]


def _relayout(tT):
    out_shape = jax.ShapeDtypeStruct((VPAD // 2, 2 * DSZ), jnp.float32)
    full = pl.pallas_call(
        _tc_body,
        grid=(NFULL,),
        in_specs=[pl.BlockSpec((DSZ, TBLK), lambda i: (0, i))],
        out_specs=pl.BlockSpec((TBLK // 2, 2 * DSZ), lambda i: (i, 0)),
        out_shape=out_shape,
    )(tT)
    tail = jnp.pad(
        lax.slice(tT, (0, NFULL * TBLK), (DSZ, VSZ)),
        ((0, 0), (0, VPAD - VSZ)),
    )
    return pl.pallas_call(
        _tc_body_tail,
        grid=(1,),
        in_specs=[
            pl.BlockSpec((DSZ, TBLK), lambda i: (0, 0)),
            pl.BlockSpec(memory_space=pl.ANY),
        ],
        out_specs=pl.BlockSpec((TBLK // 2, 2 * DSZ), lambda i: (NFULL, 0)),
        out_shape=out_shape,
        input_output_aliases={1: 0},
    )(tail, full)


def _sc_body(idx_hbm, table_hbm, out_hbm, idx1_v, idx_v, rows_v, *sems):
    gsem = sems[:NBUF]
    ssem = sems[NBUF:]
    wid = lax.axis_index("s") * NC + lax.axis_index("c")
    base = wid * PER_W
    pltpu.sync_copy(idx_hbm.at[pl.ds(base, PER_W)], idx1_v)

    @pl.loop(0, NCHUNK)
    def _(c):
        for v in range(CH // 16):
            t = idx1_v[pl.ds(c * CH + v * 16, 16)]
            m = t & (TBLK - 1)
            j = (t - m) + ((m + m) & (TBLK - 1)) + (m >> SH)
            idx_v[c, pl.ds(v * 16, 16)] = j

    for b in range(NBUF):
        pltpu.async_copy(table_hbm.at[idx_v.at[b]], rows_v.at[b], gsem[b])

    @pl.loop(0, NCHUNK - NBUF, step=NBUF)
    def _(i):
        for b in range(NBUF):
            c = i + b
            pltpu.make_async_copy(
                table_hbm.at[idx_v.at[c]], rows_v.at[b], gsem[b]
            ).wait()
            pltpu.async_copy(
                rows_v.at[b], out_hbm.at[pl.ds(base + c * CH, CH)], ssem[b]
            )
        for b in range(NBUF):
            c = i + b
            pltpu.make_async_copy(
                rows_v.at[b], out_hbm.at[pl.ds(base + c * CH, CH)], ssem[b]
            ).wait()
            pltpu.async_copy(
                table_hbm.at[idx_v.at[c + NBUF]], rows_v.at[b], gsem[b]
            )

    for b in range(NBUF):
        c = NCHUNK - NBUF + b
        pltpu.make_async_copy(
            table_hbm.at[idx_v.at[c]], rows_v.at[b], gsem[b]
        ).wait()
        pltpu.async_copy(
            rows_v.at[b], out_hbm.at[pl.ds(base + c * CH, CH)], ssem[b]
        )
    for b in range(NBUF):
        c = NCHUNK - NBUF + b
        pltpu.make_async_copy(
            rows_v.at[b], out_hbm.at[pl.ds(base + c * CH, CH)], ssem[b]
        ).wait()


def _gather(idx, table_rm):
    mesh = plsc.VectorSubcoreMesh(core_axis_name="c", subcore_axis_name="s")
    return pl.kernel(
        _sc_body,
        out_type=jax.ShapeDtypeStruct((HALF, DSZ), jnp.float32),
        mesh=mesh,
        scratch_types=[
            pltpu.VMEM((PER_W,), jnp.int32),
            pltpu.VMEM((NCHUNK, CH), jnp.int32),
            pltpu.VMEM((NBUF, CH, DSZ), jnp.float32),
        ]
        + [pltpu.SemaphoreType.DMA] * (2 * NBUF),
        compiler_params=pltpu.CompilerParams(use_tc_tiling_on_sc=False),
    )(idx, table_rm)


def _tc_post_body(in_ref, out_ref):
    out_ref[...] = in_ref[...].T


def _tc_post_body2(in_ref, full_ref, out_ref):
    del full_ref
    out_ref[...] = in_ref[...].T


def _final_transpose(flat_a, flat_b):
    nblk = B // 2 // BBLK
    out_shape = jax.ShapeDtypeStruct((L * DSZ, B), jnp.float32)
    part = pl.pallas_call(
        _tc_post_body,
        grid=(nblk,),
        in_specs=[pl.BlockSpec((BBLK, L * DSZ), lambda i: (i, 0))],
        out_specs=pl.BlockSpec((L * DSZ, BBLK), lambda i: (0, i)),
        out_shape=out_shape,
        compiler_params=pltpu.CompilerParams(needs_layout_passes=False),
    )(flat_a)
    return pl.pallas_call(
        _tc_post_body2,
        grid=(nblk,),
        in_specs=[
            pl.BlockSpec((BBLK, L * DSZ), lambda i: (i, 0)),
            pl.BlockSpec(memory_space=pl.ANY),
        ],
        out_specs=pl.BlockSpec((L * DSZ, BBLK), lambda i: (0, i + nblk)),
        out_shape=out_shape,
        input_output_aliases={1: 0},
        compiler_params=pltpu.CompilerParams(needs_layout_passes=False),
    )(flat_b, part)


@jax.jit
def _lookup(x, table):
    t2 = _relayout(table.T)
    table_rm = t2.reshape(VPAD, DSZ)
    idx = x.reshape(TOTAL)
    flat_a = _gather(idx[:HALF], table_rm)
    flat_b = _gather(idx[HALF:], table_rm)
    post = _final_transpose(
        flat_a.reshape(B // 2, L * DSZ), flat_b.reshape(B // 2, L * DSZ)
    )
    return jnp.transpose(post.reshape(L, DSZ, B), (2, 0, 1))


def kernel(x, table):
    return _lookup(x, table)


# speedup vs baseline: 1.0175x; 1.0056x over previous
import jax
import jax.numpy as jnp
from jax import lax
from jax.experimental import pallas as pl
from jax.experimental.pallas import tpu as pltpu
from jax.experimental.pallas import tpu_sc as plsc

VSZ = 1000000
DSZ = 64
B = 4096
L = 50

NC = 2
NS = 16
NW = NC * NS

TOTAL = B * L
HALF = TOTAL // 2
PER_W = HALF // NW
CH = 128
NCHUNK = PER_W // CH
NBUF = 5

TBLK = 32768
NFULL = VSZ // TBLK
VPAD = (NFULL + 1) * TBLK
SH = (TBLK // 2).bit_length() - 1

BBLK = 512


def _tc_body(tT_ref, out_ref):
    a = tT_ref[...].T
    out_ref[:, :DSZ] = a[: TBLK // 2]
    out_ref[:, DSZ:] = a[TBLK // 2 :]


def _tc_body_tail(tail_ref, full_ref, out_ref):
    del full_ref
    a = tail_ref[...].T
    out_ref[:, :DSZ] = a[: TBLK // 2]
    out_ref[:, DSZ:] = a[TBLK // 2 :]


def _relayout(tT):
    out_shape = jax.ShapeDtypeStruct((VPAD // 2, 2 * DSZ), jnp.float32)
    full = pl.pallas_call(
        _tc_body,
        grid=(NFULL,),
        in_specs=[pl.BlockSpec((DSZ, TBLK), lambda i: (0, i))],
        out_specs=pl.BlockSpec((TBLK // 2, 2 * DSZ), lambda i: (i, 0)),
        out_shape=out_shape,
    )(tT)
    tail = jnp.pad(
        lax.slice(tT, (0, NFULL * TBLK), (DSZ, VSZ)),
        ((0, 0), (0, VPAD - VSZ)),
    )
    return pl.pallas_call(
        _tc_body_tail,
        grid=(1,),
        in_specs=[
            pl.BlockSpec((DSZ, TBLK), lambda i: (0, 0)),
            pl.BlockSpec(memory_space=pl.ANY),
        ],
        out_specs=pl.BlockSpec((TBLK // 2, 2 * DSZ), lambda i: (NFULL, 0)),
        out_shape=out_shape,
        input_output_aliases={1: 0},
    )(tail, full)


def _sc_body(idx_hbm, table_hbm, out_hbm, idx1_v, idx_v, rows_v, *sems):
    gsem = sems[:NBUF]
    ssem = sems[NBUF:]
    wid = lax.axis_index("s") * NC + lax.axis_index("c")
    base = wid * PER_W
    pltpu.sync_copy(idx_hbm.at[pl.ds(base, PER_W)], idx1_v)

    @pl.loop(0, NCHUNK)
    def _(c):
        for v in range(CH // 16):
            t = idx1_v[pl.ds(c * CH + v * 16, 16)]
            m = t & (TBLK - 1)
            j = (t - m) + ((m + m) & (TBLK - 1)) + (m >> SH)
            idx_v[c, pl.ds(v * 16, 16)] = j

    for b in range(NBUF):
        pltpu.async_copy(table_hbm.at[idx_v.at[b]], rows_v.at[b], gsem[b])

    @pl.loop(0, NCHUNK - NBUF, step=NBUF)
    def _(i):
        for b in range(NBUF):
            c = i + b
            pltpu.make_async_copy(
                table_hbm.at[idx_v.at[c]], rows_v.at[b], gsem[b]
            ).wait()
            pltpu.async_copy(
                rows_v.at[b], out_hbm.at[pl.ds(base + c * CH, CH)], ssem[b]
            )
        for b in range(NBUF):
            c = i + b
            pltpu.make_async_copy(
                rows_v.at[b], out_hbm.at[pl.ds(base + c * CH, CH)], ssem[b]
            ).wait()
            pltpu.async_copy(
                table_hbm.at[idx_v.at[c + NBUF]], rows_v.at[b], gsem[b]
            )

    for b in range(NBUF):
        c = NCHUNK - NBUF + b
        pltpu.make_async_copy(
            table_hbm.at[idx_v.at[c]], rows_v.at[b], gsem[b]
        ).wait()
        pltpu.async_copy(
            rows_v.at[b], out_hbm.at[pl.ds(base + c * CH, CH)], ssem[b]
        )
    for b in range(NBUF):
        c = NCHUNK - NBUF + b
        pltpu.make_async_copy(
            rows_v.at[b], out_hbm.at[pl.ds(base + c * CH, CH)], ssem[b]
        ).wait()


def _gather(idx, table_rm):
    mesh = plsc.VectorSubcoreMesh(core_axis_name="c", subcore_axis_name="s")
    return pl.kernel(
        _sc_body,
        out_type=jax.ShapeDtypeStruct((HALF, DSZ), jnp.float32),
        mesh=mesh,
        scratch_types=[
            pltpu.VMEM((PER_W,), jnp.int32),
            pltpu.VMEM((NCHUNK, CH), jnp.int32),
            pltpu.VMEM((NBUF, CH, DSZ), jnp.float32),
        ]
        + [pltpu.SemaphoreType.DMA] * (2 * NBUF),
        compiler_params=pltpu.CompilerParams(use_tc_tiling_on_sc=False),
    )(idx, table_rm)


def _tc_post_body(in_ref, out_ref):
    out_ref[...] = in_ref[...].T


def _tc_post_body2(in_ref, full_ref, out_ref):
    del full_ref
    out_ref[...] = in_ref[...].T


def _final_transpose(flat_a, flat_b):
    nblk = B // 2 // BBLK
    out_shape = jax.ShapeDtypeStruct((L * DSZ, B), jnp.float32)
    part = pl.pallas_call(
        _tc_post_body,
        grid=(nblk,),
        in_specs=[pl.BlockSpec((BBLK, L * DSZ), lambda i: (i, 0))],
        out_specs=pl.BlockSpec((L * DSZ, BBLK), lambda i: (0, i)),
        out_shape=out_shape,
        compiler_params=pltpu.CompilerParams(needs_layout_passes=False),
    )(flat_a)
    return pl.pallas_call(
        _tc_post_body2,
        grid=(nblk,),
        in_specs=[
            pl.BlockSpec((BBLK, L * DSZ), lambda i: (i, 0)),
            pl.BlockSpec(memory_space=pl.ANY),
        ],
        out_specs=pl.BlockSpec((L * DSZ, BBLK), lambda i: (0, i + nblk)),
        out_shape=out_shape,
        input_output_aliases={1: 0},
        compiler_params=pltpu.CompilerParams(needs_layout_passes=False),
    )(flat_b, part)


@jax.jit
def _lookup(x, table):
    t2 = _relayout(table.T)
    table_rm = t2.reshape(VPAD, DSZ)
    idx = x.reshape(TOTAL)
    flat_a = _gather(idx[:HALF], table_rm)
    flat_b = _gather(idx[HALF:], table_rm)
    post = _final_transpose(
        flat_a.reshape(B // 2, L * DSZ), flat_b.reshape(B // 2, L * DSZ)
    )
    return jnp.transpose(post.reshape(L, DSZ, B), (2, 0, 1))


def kernel(x, table):
    return _lookup(x, table)


# speedup vs baseline: 1.0307x; 1.0130x over previous
import jax
import jax.numpy as jnp
from jax import lax
from jax.experimental import pallas as pl
from jax.experimental.pallas import tpu as pltpu
from jax.experimental.pallas import tpu_sc as plsc

VSZ = 1000000
DSZ = 64
B = 4096
L = 50

NC = 2
NS = 16
NW = NC * NS

TOTAL = B * L
HALF = TOTAL // 2
PER_W = HALF // NW
CH = 128
NCHUNK = PER_W // CH
NBUF = 5

TBLK = 32768
NFULL = VSZ // TBLK
VPAD = (NFULL + 1) * TBLK
SH = (TBLK // 2).bit_length() - 1

BBLK = 1024


def _tc_body(tT_ref, out_ref):
    a = tT_ref[...].T
    out_ref[:, :DSZ] = a[: TBLK // 2]
    out_ref[:, DSZ:] = a[TBLK // 2 :]


def _tc_body_tail(tail_ref, full_ref, out_ref):
    del full_ref
    a = tail_ref[...].T
    out_ref[:, :DSZ] = a[: TBLK // 2]
    out_ref[:, DSZ:] = a[TBLK // 2 :]


def _relayout(tT):
    out_shape = jax.ShapeDtypeStruct((VPAD // 2, 2 * DSZ), jnp.float32)
    full = pl.pallas_call(
        _tc_body,
        grid=(NFULL,),
        in_specs=[pl.BlockSpec((DSZ, TBLK), lambda i: (0, i))],
        out_specs=pl.BlockSpec((TBLK // 2, 2 * DSZ), lambda i: (i, 0)),
        out_shape=out_shape,
    )(tT)
    tail = jnp.pad(
        lax.slice(tT, (0, NFULL * TBLK), (DSZ, VSZ)),
        ((0, 0), (0, VPAD - VSZ)),
    )
    return pl.pallas_call(
        _tc_body_tail,
        grid=(1,),
        in_specs=[
            pl.BlockSpec((DSZ, TBLK), lambda i: (0, 0)),
            pl.BlockSpec(memory_space=pl.ANY),
        ],
        out_specs=pl.BlockSpec((TBLK // 2, 2 * DSZ), lambda i: (NFULL, 0)),
        out_shape=out_shape,
        input_output_aliases={1: 0},
    )(tail, full)


def _sc_body(idx_hbm, table_hbm, out_hbm, idx1_v, idx_v, rows_v, *sems):
    gsem = sems[:NBUF]
    ssem = sems[NBUF:]
    wid = lax.axis_index("s") * NC + lax.axis_index("c")
    base = wid * PER_W
    pltpu.sync_copy(idx_hbm.at[pl.ds(base, PER_W)], idx1_v)

    @pl.loop(0, NCHUNK)
    def _(c):
        for v in range(CH // 16):
            t = idx1_v[pl.ds(c * CH + v * 16, 16)]
            m = t & (TBLK - 1)
            j = (t - m) + ((m + m) & (TBLK - 1)) + (m >> SH)
            idx_v[c, pl.ds(v * 16, 16)] = j

    for b in range(NBUF):
        pltpu.async_copy(table_hbm.at[idx_v.at[b]], rows_v.at[b], gsem[b])

    @pl.loop(0, NCHUNK - NBUF, step=NBUF)
    def _(i):
        for b in range(NBUF):
            c = i + b
            pltpu.make_async_copy(
                table_hbm.at[idx_v.at[c]], rows_v.at[b], gsem[b]
            ).wait()
            pltpu.async_copy(
                rows_v.at[b], out_hbm.at[pl.ds(base + c * CH, CH)], ssem[b]
            )
        for b in range(NBUF):
            c = i + b
            pltpu.make_async_copy(
                rows_v.at[b], out_hbm.at[pl.ds(base + c * CH, CH)], ssem[b]
            ).wait()
            pltpu.async_copy(
                table_hbm.at[idx_v.at[c + NBUF]], rows_v.at[b], gsem[b]
            )

    for b in range(NBUF):
        c = NCHUNK - NBUF + b
        pltpu.make_async_copy(
            table_hbm.at[idx_v.at[c]], rows_v.at[b], gsem[b]
        ).wait()
        pltpu.async_copy(
            rows_v.at[b], out_hbm.at[pl.ds(base + c * CH, CH)], ssem[b]
        )
    for b in range(NBUF):
        c = NCHUNK - NBUF + b
        pltpu.make_async_copy(
            rows_v.at[b], out_hbm.at[pl.ds(base + c * CH, CH)], ssem[b]
        ).wait()


def _gather(idx, table_rm):
    mesh = plsc.VectorSubcoreMesh(core_axis_name="c", subcore_axis_name="s")
    return pl.kernel(
        _sc_body,
        out_type=jax.ShapeDtypeStruct((HALF, DSZ), jnp.float32),
        mesh=mesh,
        scratch_types=[
            pltpu.VMEM((PER_W,), jnp.int32),
            pltpu.VMEM((NCHUNK, CH), jnp.int32),
            pltpu.VMEM((NBUF, CH, DSZ), jnp.float32),
        ]
        + [pltpu.SemaphoreType.DMA] * (2 * NBUF),
        compiler_params=pltpu.CompilerParams(use_tc_tiling_on_sc=False),
    )(idx, table_rm)


def _tc_post_body(in_ref, out_ref):
    out_ref[...] = in_ref[...].T


def _tc_post_body2(in_ref, full_ref, out_ref):
    del full_ref
    out_ref[...] = in_ref[...].T


def _final_transpose(flat_a, flat_b):
    nblk = B // 2 // BBLK
    out_shape = jax.ShapeDtypeStruct((L * DSZ, B), jnp.float32)
    part = pl.pallas_call(
        _tc_post_body,
        grid=(nblk,),
        in_specs=[pl.BlockSpec((BBLK, L * DSZ), lambda i: (i, 0))],
        out_specs=pl.BlockSpec((L * DSZ, BBLK), lambda i: (0, i)),
        out_shape=out_shape,
        compiler_params=pltpu.CompilerParams(needs_layout_passes=False),
    )(flat_a)
    return pl.pallas_call(
        _tc_post_body2,
        grid=(nblk,),
        in_specs=[
            pl.BlockSpec((BBLK, L * DSZ), lambda i: (i, 0)),
            pl.BlockSpec(memory_space=pl.ANY),
        ],
        out_specs=pl.BlockSpec((L * DSZ, BBLK), lambda i: (0, i + nblk)),
        out_shape=out_shape,
        input_output_aliases={1: 0},
        compiler_params=pltpu.CompilerParams(needs_layout_passes=False),
    )(flat_b, part)


@jax.jit
def _lookup(x, table):
    t2 = _relayout(table.T)
    table_rm = t2.reshape(VPAD, DSZ)
    idx = x.reshape(TOTAL)
    flat_a = _gather(idx[:HALF], table_rm)
    flat_b = _gather(idx[HALF:], table_rm)
    post = _final_transpose(
        flat_a.reshape(B // 2, L * DSZ), flat_b.reshape(B // 2, L * DSZ)
    )
    return jnp.transpose(post.reshape(L, DSZ, B), (2, 0, 1))


def kernel(x, table):
    return _lookup(x, table)
